# Initial kernel scaffold; baseline (speedup 1.0000x reference)
import jax
import jax.numpy as jnp
from jax.experimental import pallas as pl


def kernel(x, edge_index, edge_attr, W1, b1, W2, b2):
    raise NotImplementedError("write your pallas kernel here")


# speedup vs baseline: 9.0993x; 9.0993x over previous
import dataclasses
import functools

import jax
import jax.numpy as jnp
from jax import lax
from jax.experimental import pallas as pl
from jax.experimental.pallas import tpu as pltpu
from jax.experimental.pallas import tpu_sc as plsc

NN = 10000
EE = 320000
DD = 128
NC = 2
NS = 16
NW = NC * NS
EPW = EE // NW
CH = 80
NCHUNK = EPW // CH
NPAD = 10240
RPT = NPAD // NS

_vmesh = plsc.VectorSubcoreMesh(core_axis_name="c", subcore_axis_name="s")

_sc_params = pltpu.CompilerParams()
if "needs_layout_passes" in pltpu.CompilerParams.__dataclass_fields__:
    _sc_params = dataclasses.replace(_sc_params, needs_layout_passes=False)


def _sc_deg_body(col_hbm, ew_hbm, out_hbm, colv, ewv, zv, degsh):
    cid = lax.axis_index("c")
    sid = lax.axis_index("s")
    wid = sid * NC + cid

    @pl.loop(0, RPT, step=16)
    def _zero(i):
        zv[pl.ds(i, 16)] = jnp.zeros((16,), jnp.float32)

    pltpu.sync_copy(zv, degsh.at[pl.ds(sid * RPT, RPT)])
    plsc.subcore_barrier()

    base = wid * EPW

    @pl.loop(0, NCHUNK)
    def _chunk(c):
        off = base + c * CH
        pltpu.sync_copy(col_hbm.at[pl.ds(off, CH)], colv)
        pltpu.sync_copy(ew_hbm.at[pl.ds(off, CH)], ewv)
        pltpu.sync_copy(ewv, degsh.at[colv], add=True)

    plsc.subcore_barrier()
    pltpu.sync_copy(degsh.at[pl.ds(sid * RPT, RPT)],
                    out_hbm.at[cid, pl.ds(sid * RPT, RPT)])


def _sc_deg(col, ew):
    k = pl.kernel(
        _sc_deg_body,
        out_type=jax.ShapeDtypeStruct((NC, NPAD), jnp.float32),
        mesh=_vmesh,
        scratch_types=[
            pltpu.VMEM((CH,), jnp.int32),
            pltpu.VMEM((CH,), jnp.float32),
            pltpu.VMEM((RPT,), jnp.float32),
            pltpu.VMEM_SHARED((NPAD,), jnp.float32),
        ],
    )
    return k(col, ew)


def _sc_msg_body(hp_hbm, row_hbm, col_hbm, ew_hbm, out_hbm,
                 rowv, colv, ewv, rows, zrows, ssh, sem):
    cid = lax.axis_index("c")
    sid = lax.axis_index("s")
    wid = sid * NC + cid

    @pl.loop(0, 128)
    def _zero(r):
        for p in range(DD // 16):
            zrows[r, pl.ds(p * 16, 16)] = jnp.zeros((16,), jnp.float32)

    for z in range(RPT // 128):
        pltpu.sync_copy(zrows, ssh.at[pl.ds(sid * RPT + z * 128, 128)])
    plsc.subcore_barrier()

    base = wid * EPW

    @pl.loop(0, NCHUNK)
    def _chunk(c):
        off = base + c * CH
        pltpu.sync_copy(row_hbm.at[pl.ds(off, CH)], rowv)
        pltpu.sync_copy(col_hbm.at[pl.ds(off, CH)], colv)
        pltpu.sync_copy(ew_hbm.at[pl.ds(off, CH)], ewv)
        pltpu.async_copy(hp_hbm.at[rowv], rows, sem).wait()

        @pl.loop(0, CH)
        def _scale(j):
            w = plsc.load_gather(ewv, [jnp.zeros((16,), jnp.int32) + j])
            for p in range(DD // 16):
                sl = pl.ds(p * 16, 16)
                rows[j, sl] = rows[j, sl] * w

        pltpu.sync_copy(rows, ssh.at[colv], add=True)

    plsc.subcore_barrier()
    for z in range(RPT // 128):
        r0 = sid * RPT + z * 128
        pltpu.sync_copy(ssh.at[pl.ds(r0, 128)], out_hbm.at[cid, pl.ds(r0, 128)])


def _sc_msg(hp, row, col, ew):
    k = pl.kernel(
        _sc_msg_body,
        out_type=jax.ShapeDtypeStruct((NC, NPAD, DD), jnp.float32),
        mesh=_vmesh,
        scratch_types=[
            pltpu.VMEM((CH,), jnp.int32),
            pltpu.VMEM((CH,), jnp.int32),
            pltpu.VMEM((CH,), jnp.float32),
            pltpu.VMEM((CH, DD), jnp.float32),
            pltpu.VMEM((128, DD), jnp.float32),
            pltpu.VMEM_SHARED((NPAD, DD), jnp.float32),
            pltpu.SemaphoreType.DMA,
        ],
        compiler_params=_sc_params,
    )
    return k(hp, row, col, ew)


_RB = 1000
_GRID = NN // _RB


def _tc_first_body(x_ref, w_ref, dgt_ref, hp_ref, dinv_ref):
    deg = 1.0 + dgt_ref[:, 0] + dgt_ref[:, 1]
    dinv = lax.rsqrt(deg)[:, None]
    dinv_ref[...] = dinv
    h = jnp.dot(x_ref[...], w_ref[...], preferred_element_type=jnp.float32)
    hp_ref[...] = h * dinv


def _tc_first(x, W1, dgt):
    return pl.pallas_call(
        _tc_first_body,
        grid=(_GRID,),
        in_specs=[
            pl.BlockSpec((_RB, DD), lambda i: (i, 0)),
            pl.BlockSpec((DD, DD), lambda i: (0, 0)),
            pl.BlockSpec((_RB, NC), lambda i: (i, 0)),
        ],
        out_specs=[
            pl.BlockSpec((_RB, DD), lambda i: (i, 0)),
            pl.BlockSpec((_RB, 1), lambda i: (i, 0)),
        ],
        out_shape=[
            jax.ShapeDtypeStruct((NN, DD), jnp.float32),
            jax.ShapeDtypeStruct((NN, 1), jnp.float32),
        ],
    )(x, W1, dgt)


def _tc_mid_body(sp_ref, hp_ref, dinv_ref, b_ref, w_ref, out_ref):
    s = sp_ref[0] + sp_ref[1] + hp_ref[...]
    z = jnp.maximum(dinv_ref[...] * s + b_ref[...], 0.0)
    h = jnp.dot(z, w_ref[...], preferred_element_type=jnp.float32)
    out_ref[...] = h * dinv_ref[...]


def _tc_mid(sp, hp, dinv, b, W2):
    return pl.pallas_call(
        _tc_mid_body,
        grid=(_GRID,),
        in_specs=[
            pl.BlockSpec((NC, _RB, DD), lambda i: (0, i, 0)),
            pl.BlockSpec((_RB, DD), lambda i: (i, 0)),
            pl.BlockSpec((_RB, 1), lambda i: (i, 0)),
            pl.BlockSpec((1, DD), lambda i: (0, 0)),
            pl.BlockSpec((DD, DD), lambda i: (0, 0)),
        ],
        out_specs=pl.BlockSpec((_RB, DD), lambda i: (i, 0)),
        out_shape=jax.ShapeDtypeStruct((NN, DD), jnp.float32),
    )(sp, hp, dinv, b, W2)


def _tc_last_body(sp_ref, hp_ref, dinv_ref, b_ref, out_ref):
    s = sp_ref[0] + sp_ref[1] + hp_ref[...]
    out_ref[...] = dinv_ref[...] * s + b_ref[...]


def _tc_last(sp, hp, dinv, b):
    return pl.pallas_call(
        _tc_last_body,
        grid=(_GRID,),
        in_specs=[
            pl.BlockSpec((NC, _RB, DD), lambda i: (0, i, 0)),
            pl.BlockSpec((_RB, DD), lambda i: (i, 0)),
            pl.BlockSpec((_RB, 1), lambda i: (i, 0)),
            pl.BlockSpec((1, DD), lambda i: (0, 0)),
        ],
        out_specs=pl.BlockSpec((_RB, DD), lambda i: (i, 0)),
        out_shape=jax.ShapeDtypeStruct((NN, DD), jnp.float32),
    )(sp, hp, dinv, b)


def kernel(x, edge_index, edge_attr, W1, b1, W2, b2):
    row = edge_index[0]
    col = edge_index[1]
    degp = _sc_deg(col, edge_attr)
    dgt = jnp.transpose(degp[:, :NN])
    hp1, dinv = _tc_first(x, W1, dgt)
    s1 = _sc_msg(hp1, row, col, edge_attr)
    hp2 = _tc_mid(s1[:, :NN], hp1, dinv, b1.reshape(1, DD), W2)
    s2 = _sc_msg(hp2, row, col, edge_attr)
    out = _tc_last(s2[:, :NN], hp2, dinv, b2.reshape(1, DD))
    return out


# speedup vs baseline: 19.7698x; 2.1727x over previous
import dataclasses
import functools

import jax
import jax.numpy as jnp
from jax import lax
from jax.experimental import pallas as pl
from jax.experimental.pallas import tpu as pltpu
from jax.experimental.pallas import tpu_sc as plsc

NN = 10000
EE = 320000
DD = 128
NC = 2
NS = 16
NW = NC * NS
EPW = EE // NW
CH = 80
NCHUNK = EPW // CH
NPAD = 10240
RPT = NPAD // NS

_vmesh = plsc.VectorSubcoreMesh(core_axis_name="c", subcore_axis_name="s")

_sc_params = pltpu.CompilerParams()
if "needs_layout_passes" in pltpu.CompilerParams.__dataclass_fields__:
    _sc_params = dataclasses.replace(_sc_params, needs_layout_passes=False)


def _splat16(v):
    return jnp.zeros((16,), jnp.int32) + v


def _vcopy_idx(src1d, base, dst1d, n):
    q = 0
    while q + 16 <= n:
        dst1d[pl.ds(q, 16)] = src1d[pl.ds(base + q, 16)]
        q += 16
    if q < n:
        dst1d[pl.ds(n - 16, 16)] = src1d[pl.ds(base + n - 16, 16)]


def _sc_deg_body(col_hbm, ew_hbm, out_hbm, colall, ewall, zv, degsh, sem):
    cid = lax.axis_index("c")
    sid = lax.axis_index("s")
    wid = sid * NC + cid

    @pl.loop(0, RPT, step=16)
    def _zero(i):
        zv[pl.ds(i, 16)] = jnp.zeros((16,), jnp.float32)

    pltpu.sync_copy(zv, degsh.at[pl.ds(sid * RPT, RPT)])
    pltpu.sync_copy(col_hbm.at[wid], colall)
    pltpu.sync_copy(ew_hbm.at[wid], ewall)
    plsc.subcore_barrier()

    @pl.loop(0, NCHUNK // 5)
    def _chunk(g):
        descs = []
        for u in range(5):
            c = g * 5 + u
            descs.append(pltpu.async_copy(
                ewall.at[c], degsh.at[colall.at[c]], sem, add=True))
        for d in descs:
            d.wait()

    plsc.subcore_barrier()
    pltpu.sync_copy(degsh.at[pl.ds(sid * RPT, RPT)],
                    out_hbm.at[cid, pl.ds(sid * RPT, RPT)])


def _sc_deg(col3, ew3):
    k = pl.kernel(
        _sc_deg_body,
        out_type=jax.ShapeDtypeStruct((NC, NPAD), jnp.float32),
        mesh=_vmesh,
        scratch_types=[
            pltpu.VMEM((NCHUNK, CH), jnp.int32),
            pltpu.VMEM((NCHUNK, CH), jnp.float32),
            pltpu.VMEM((RPT,), jnp.float32),
            pltpu.VMEM_SHARED((NPAD,), jnp.float32),
            pltpu.SemaphoreType.DMA,
        ],
        compiler_params=_sc_params,
    )
    return k(col3, ew3)


CHM = 40
NCHM = EPW // CHM


def _sc_msg_body(hp_hbm, row_hbm, col_hbm, ew_hbm, out_hbm,
                 rowall, colall, ewall, colv, buf0, buf1, ssh, sem0, sem1):
    cid = lax.axis_index("c")
    sid = lax.axis_index("s")
    wid = sid * NC + cid

    @pl.loop(0, CHM)
    def _zero(r):
        for p in range(DD // 16):
            buf0[r, pl.ds(p * 16, 16)] = jnp.zeros((16,), jnp.float32)

    for z in range(RPT // CHM):
        pltpu.sync_copy(buf0, ssh.at[pl.ds(sid * RPT + z * CHM, CHM)])
    base = wid * EPW
    pltpu.sync_copy(row_hbm.at[pl.ds(base, EPW)], rowall)
    pltpu.sync_copy(col_hbm.at[pl.ds(base, EPW)], colall)
    pltpu.sync_copy(ew_hbm.at[pl.ds(base, EPW)], ewall)
    plsc.subcore_barrier()

    def scale(buf, c):
        @pl.loop(0, CHM)
        def _s(j):
            w = plsc.load_gather(ewall, [_splat16(c * CHM + j)])
            for p in range(DD // 16):
                sl = pl.ds(p * 16, 16)
                buf[j, sl] = buf[j, sl] * w

    def process(buf, c):
        scale(buf, c)
        _vcopy_idx(colall, c * CHM, colv, CHM)
        pltpu.sync_copy(buf, ssh.at[colv], add=True)

    def gather(buf, sem, c):
        return pltpu.async_copy(hp_hbm.at[rowall.at[pl.ds(c * CHM, CHM)]],
                                buf, sem)

    def gwait(buf, sem, c):
        pltpu.make_async_copy(hp_hbm.at[rowall.at[pl.ds(c * CHM, CHM)]],
                              buf, sem).wait()

    gather(buf0, sem0, 0)

    @pl.loop(0, NCHM - 2, step=2)
    def _chunk(c):
        gather(buf1, sem1, c + 1)
        gwait(buf0, sem0, c)
        process(buf0, c)
        gather(buf0, sem0, c + 2)
        gwait(buf1, sem1, c + 1)
        process(buf1, c + 1)

    gather(buf1, sem1, NCHM - 1)
    gwait(buf0, sem0, NCHM - 2)
    process(buf0, NCHM - 2)
    gwait(buf1, sem1, NCHM - 1)
    process(buf1, NCHM - 1)

    plsc.subcore_barrier()
    for z in range(RPT // 128):
        r0 = sid * RPT + z * 128
        pltpu.sync_copy(ssh.at[pl.ds(r0, 128)], out_hbm.at[cid, pl.ds(r0, 128)])


def _sc_msg(hp, row, col, ew):
    k = pl.kernel(
        _sc_msg_body,
        out_type=jax.ShapeDtypeStruct((NC, NPAD, DD), jnp.float32),
        mesh=_vmesh,
        scratch_types=[
            pltpu.VMEM((EPW,), jnp.int32),
            pltpu.VMEM((EPW,), jnp.int32),
            pltpu.VMEM((EPW,), jnp.float32),
            pltpu.VMEM((CHM,), jnp.int32),
            pltpu.VMEM((CHM, DD), jnp.float32),
            pltpu.VMEM((CHM, DD), jnp.float32),
            pltpu.VMEM_SHARED((NPAD, DD), jnp.float32),
            pltpu.SemaphoreType.DMA,
            pltpu.SemaphoreType.DMA,
        ],
        compiler_params=_sc_params,
    )
    return k(hp, row, col, ew)


_RB = 1000
_GRID = NN // _RB


def _tc_mm_body(x_ref, w_ref, h_ref):
    h_ref[...] = jnp.dot(x_ref[...], w_ref[...],
                         preferred_element_type=jnp.float32)


def _tc_mm(x, W):
    return pl.pallas_call(
        _tc_mm_body,
        grid=(_GRID,),
        in_specs=[
            pl.BlockSpec((_RB, DD), lambda i: (i, 0)),
            pl.BlockSpec((DD, DD), lambda i: (0, 0)),
        ],
        out_specs=pl.BlockSpec((_RB, DD), lambda i: (i, 0)),
        out_shape=jax.ShapeDtypeStruct((NN, DD), jnp.float32),
    )(x, W)


def _tc_scale_body(h_ref, dgt_ref, hp_ref, dinv_ref):
    deg = 1.0 + dgt_ref[:, 0] + dgt_ref[:, 1]
    dinv = lax.rsqrt(deg)[:, None]
    dinv_ref[...] = dinv
    hp_ref[...] = h_ref[...] * dinv


def _tc_scale(h, dgt):
    return pl.pallas_call(
        _tc_scale_body,
        grid=(_GRID,),
        in_specs=[
            pl.BlockSpec((_RB, DD), lambda i: (i, 0)),
            pl.BlockSpec((_RB, NC), lambda i: (i, 0)),
        ],
        out_specs=[
            pl.BlockSpec((_RB, DD), lambda i: (i, 0)),
            pl.BlockSpec((_RB, 1), lambda i: (i, 0)),
        ],
        out_shape=[
            jax.ShapeDtypeStruct((NN, DD), jnp.float32),
            jax.ShapeDtypeStruct((NN, 1), jnp.float32),
        ],
    )(h, dgt)


def _tc_mid_body(sp_ref, hp_ref, dinv_ref, b_ref, w_ref, out_ref):
    s = sp_ref[0] + sp_ref[1] + hp_ref[...]
    z = jnp.maximum(dinv_ref[...] * s + b_ref[...], 0.0)
    h = jnp.dot(z, w_ref[...], preferred_element_type=jnp.float32)
    out_ref[...] = h * dinv_ref[...]


def _tc_mid(sp, hp, dinv, b, W2):
    return pl.pallas_call(
        _tc_mid_body,
        grid=(_GRID,),
        in_specs=[
            pl.BlockSpec((NC, _RB, DD), lambda i: (0, i, 0)),
            pl.BlockSpec((_RB, DD), lambda i: (i, 0)),
            pl.BlockSpec((_RB, 1), lambda i: (i, 0)),
            pl.BlockSpec((1, DD), lambda i: (0, 0)),
            pl.BlockSpec((DD, DD), lambda i: (0, 0)),
        ],
        out_specs=pl.BlockSpec((_RB, DD), lambda i: (i, 0)),
        out_shape=jax.ShapeDtypeStruct((NN, DD), jnp.float32),
    )(sp, hp, dinv, b, W2)


def _tc_last_body(sp_ref, hp_ref, dinv_ref, b_ref, out_ref):
    s = sp_ref[0] + sp_ref[1] + hp_ref[...]
    out_ref[...] = dinv_ref[...] * s + b_ref[...]


def _tc_last(sp, hp, dinv, b):
    return pl.pallas_call(
        _tc_last_body,
        grid=(_GRID,),
        in_specs=[
            pl.BlockSpec((NC, _RB, DD), lambda i: (0, i, 0)),
            pl.BlockSpec((_RB, DD), lambda i: (i, 0)),
            pl.BlockSpec((_RB, 1), lambda i: (i, 0)),
            pl.BlockSpec((1, DD), lambda i: (0, 0)),
        ],
        out_specs=pl.BlockSpec((_RB, DD), lambda i: (i, 0)),
        out_shape=jax.ShapeDtypeStruct((NN, DD), jnp.float32),
    )(sp, hp, dinv, b)


def kernel(x, edge_index, edge_attr, W1, b1, W2, b2):
    row = edge_index[0]
    col = edge_index[1]
    col3 = col.reshape(NW, NCHUNK, CH)
    ew3 = edge_attr.reshape(NW, NCHUNK, CH)
    degp = _sc_deg(col3, ew3)
    h1 = _tc_mm(x, W1)
    dgt = jnp.transpose(degp[:, :NN])
    hp1, dinv = _tc_scale(h1, dgt)
    s1 = _sc_msg(hp1, row, col, edge_attr)
    hp2 = _tc_mid(s1[:, :NN], hp1, dinv, b1.reshape(1, DD), W2)
    s2 = _sc_msg(hp2, row, col, edge_attr)
    out = _tc_last(s2[:, :NN], hp2, dinv, b2.reshape(1, DD))
    return out


# speedup vs baseline: 28.2323x; 1.4281x over previous
import dataclasses
import functools

import jax
import jax.numpy as jnp
from jax import lax
from jax.experimental import pallas as pl
from jax.experimental.pallas import tpu as pltpu
from jax.experimental.pallas import tpu_sc as plsc

NN = 10000
EE = 320000
DD = 128
NC = 2
NS = 16
NW = NC * NS
EPW = EE // NW
CH = 80
NCHUNK = EPW // CH
NPAD = 10240
RPT = NPAD // NS
NPADM = 10112
RPTM = NPADM // NS

_vmesh = plsc.VectorSubcoreMesh(core_axis_name="c", subcore_axis_name="s")

_sc_params = pltpu.CompilerParams()
if "needs_layout_passes" in pltpu.CompilerParams.__dataclass_fields__:
    _sc_params = dataclasses.replace(_sc_params, needs_layout_passes=False)


def _splat16(v):
    return jnp.zeros((16,), jnp.int32) + v


def _vcopy_idx(src1d, base, dst1d, n):
    q = 0
    while q + 16 <= n:
        dst1d[pl.ds(q, 16)] = src1d[pl.ds(base + q, 16)]
        q += 16
    if q < n:
        dst1d[pl.ds(n - 16, 16)] = src1d[pl.ds(base + n - 16, 16)]


def _sc_deg_body(col_hbm, ew_hbm, out_hbm, colall, ewall, zv, degsh, sem):
    cid = lax.axis_index("c")
    sid = lax.axis_index("s")
    wid = sid * NC + cid

    @pl.loop(0, RPT, step=16)
    def _zero(i):
        zv[pl.ds(i, 16)] = jnp.zeros((16,), jnp.float32)

    pltpu.sync_copy(zv, degsh.at[pl.ds(sid * RPT, RPT)])
    pltpu.sync_copy(col_hbm.at[wid], colall)
    pltpu.sync_copy(ew_hbm.at[wid], ewall)
    plsc.subcore_barrier()

    @pl.loop(0, NCHUNK // 5)
    def _chunk(g):
        descs = []
        for u in range(5):
            c = g * 5 + u
            descs.append(pltpu.async_copy(
                ewall.at[c], degsh.at[colall.at[c]], sem, add=True))
        for d in descs:
            d.wait()

    plsc.subcore_barrier()
    pltpu.sync_copy(degsh.at[pl.ds(sid * RPT, RPT)],
                    out_hbm.at[cid, pl.ds(sid * RPT, RPT)])


def _sc_deg(col3, ew3):
    k = pl.kernel(
        _sc_deg_body,
        out_type=jax.ShapeDtypeStruct((NC, NPAD), jnp.float32),
        mesh=_vmesh,
        scratch_types=[
            pltpu.VMEM((NCHUNK, CH), jnp.int32),
            pltpu.VMEM((NCHUNK, CH), jnp.float32),
            pltpu.VMEM((RPT,), jnp.float32),
            pltpu.VMEM_SHARED((NPAD,), jnp.float32),
            pltpu.SemaphoreType.DMA,
        ],
        compiler_params=_sc_params,
    )
    return k(col3, ew3)


CHM = 80
NCHM = EPW // CHM


def _sc_msg_body(hp_hbm, row_hbm, col_hbm, ew_hbm, out_hbm,
                 rowvs, colvs, ewvs, scolvs, bufs, ssh, gsems, ssems, isems):
    cid = lax.axis_index("c")
    sid = lax.axis_index("s")
    wid = sid * NC + cid
    base = wid * EPW

    for b in range(4):
        @pl.loop(0, CHM)
        def _zero(r, _b=b):
            for p in range(DD // 16):
                bufs[_b][r, pl.ds(p * 16, 16)] = jnp.zeros((16,), jnp.float32)

        for q in range(CHM // 16):
            scolvs[b][pl.ds(q * 16, 16)] = jnp.zeros((16,), jnp.int32)

    for z in range(RPTM // CHM):
        pltpu.sync_copy(bufs[0], ssh.at[pl.ds(sid * RPTM + z * CHM, CHM)])
    rem = RPTM % CHM
    if rem:
        pltpu.sync_copy(
            bufs[0].at[pl.ds(0, rem)],
            ssh.at[pl.ds(sid * RPTM + (RPTM // CHM) * CHM, rem)])
    plsc.subcore_barrier()

    def iload(c, k):
        sl = pl.ds(base + c * CHM, CHM)
        pltpu.async_copy(row_hbm.at[sl], rowvs[k], isems[k])
        pltpu.async_copy(col_hbm.at[sl], colvs[k], isems[k])
        pltpu.async_copy(ew_hbm.at[sl], ewvs[k], isems[k])

    def iwait(c, k):
        sl = pl.ds(base + c * CHM, CHM)
        pltpu.make_async_copy(row_hbm.at[sl], rowvs[k], isems[k]).wait()
        pltpu.make_async_copy(col_hbm.at[sl], colvs[k], isems[k]).wait()
        pltpu.make_async_copy(ew_hbm.at[sl], ewvs[k], isems[k]).wait()

    def gather(k):
        pltpu.async_copy(hp_hbm.at[rowvs[k]], bufs[k], gsems[k])

    def gwait(k):
        pltpu.make_async_copy(hp_hbm.at[rowvs[k]], bufs[k], gsems[k]).wait()

    def swait(k):
        pltpu.make_async_copy(bufs[k], ssh.at[scolvs[k]], ssems[k]).wait()

    def scale(buf, ewv):
        @pl.loop(0, CHM, step=2)
        def _s(j):
            w0 = plsc.load_gather(ewv, [_splat16(j)])
            w1 = plsc.load_gather(ewv, [_splat16(j + 1)])
            for p in range(DD // 16):
                sl = pl.ds(p * 16, 16)
                buf[j, sl] = buf[j, sl] * w0
            for p in range(DD // 16):
                sl = pl.ds(p * 16, 16)
                buf[j + 1, sl] = buf[j + 1, sl] * w1

    def quarter(c, k, do_gather, do_iload):
        k2 = (k + 2) % 4
        gwait(k)
        scale(bufs[k], ewvs[k])
        swait(k2)
        if do_gather:
            iwait(c + 2, k2)
            gather(k2)
        _vcopy_idx(colvs[k], 0, scolvs[k], CHM)
        pltpu.async_copy(bufs[k], ssh.at[scolvs[k]], ssems[k], add=True)
        if do_iload:
            iload(c + 4, k)

    for k in range(4):
        iload(k, k)
    iwait(0, 0)
    gather(0)
    iwait(1, 1)
    gather(1)
    pltpu.async_copy(bufs[2], ssh.at[scolvs[2]], ssems[2], add=True)
    pltpu.async_copy(bufs[3], ssh.at[scolvs[3]], ssems[3], add=True)

    @pl.loop(0, NCHM - 5, step=4)
    def _chunk(c0):
        for q in range(4):
            quarter(c0 + q, q, True, True)

    for c in range(NCHM - 5, NCHM):
        quarter(c, c % 4, c + 2 < NCHM, c + 4 < NCHM)

    swait((NCHM - 2) % 4)
    swait((NCHM - 1) % 4)

    plsc.subcore_barrier()
    for z in range(RPTM // 128):
        r0 = sid * RPTM + z * 128
        pltpu.sync_copy(ssh.at[pl.ds(r0, 128)], out_hbm.at[cid, pl.ds(r0, 128)])
    remr = RPTM % 128
    if remr:
        r0 = sid * RPTM + (RPTM // 128) * 128
        pltpu.sync_copy(ssh.at[pl.ds(r0, remr)],
                        out_hbm.at[cid, pl.ds(r0, remr)])


def _sc_msg(hp, row, col, ew):
    def body(hp_hbm, row_hbm, col_hbm, ew_hbm, out_hbm,
             rv0, rv1, rv2, rv3, cv0, cv1, cv2, cv3, ev0, ev1, ev2, ev3,
             sv0, sv1, sv2, sv3, b0, b1, b2, b3, ssh,
             gs0, gs1, gs2, gs3, ss0, ss1, ss2, ss3, is0, is1, is2, is3):
        _sc_msg_body(hp_hbm, row_hbm, col_hbm, ew_hbm, out_hbm,
                     [rv0, rv1, rv2, rv3], [cv0, cv1, cv2, cv3],
                     [ev0, ev1, ev2, ev3], [sv0, sv1, sv2, sv3],
                     [b0, b1, b2, b3], ssh,
                     [gs0, gs1, gs2, gs3], [ss0, ss1, ss2, ss3],
                     [is0, is1, is2, is3])

    k = pl.kernel(
        body,
        out_type=jax.ShapeDtypeStruct((NC, NPADM, DD), jnp.float32),
        mesh=_vmesh,
        scratch_types=(
            [pltpu.VMEM((CHM,), jnp.int32) for _ in range(4)]
            + [pltpu.VMEM((CHM,), jnp.int32) for _ in range(4)]
            + [pltpu.VMEM((CHM,), jnp.float32) for _ in range(4)]
            + [pltpu.VMEM((CHM,), jnp.int32) for _ in range(4)]
            + [pltpu.VMEM((CHM, DD), jnp.float32) for _ in range(4)]
            + [pltpu.VMEM_SHARED((NPADM, DD), jnp.float32)]
            + [pltpu.SemaphoreType.DMA for _ in range(12)]
        ),
        compiler_params=_sc_params,
    )
    return k(hp, row, col, ew)


_RB = 1000
_GRID = NN // _RB


def _tc_mm_body(x_ref, w_ref, h_ref):
    h_ref[...] = jnp.dot(x_ref[...], w_ref[...],
                         preferred_element_type=jnp.float32)


def _tc_mm(x, W):
    return pl.pallas_call(
        _tc_mm_body,
        grid=(_GRID,),
        in_specs=[
            pl.BlockSpec((_RB, DD), lambda i: (i, 0)),
            pl.BlockSpec((DD, DD), lambda i: (0, 0)),
        ],
        out_specs=pl.BlockSpec((_RB, DD), lambda i: (i, 0)),
        out_shape=jax.ShapeDtypeStruct((NN, DD), jnp.float32),
    )(x, W)


def _tc_scale_body(h_ref, dgt_ref, hp_ref, dinv_ref):
    deg = 1.0 + dgt_ref[:, 0] + dgt_ref[:, 1]
    dinv = lax.rsqrt(deg)[:, None]
    dinv_ref[...] = dinv
    hp_ref[...] = h_ref[...] * dinv


def _tc_scale(h, dgt):
    return pl.pallas_call(
        _tc_scale_body,
        grid=(_GRID,),
        in_specs=[
            pl.BlockSpec((_RB, DD), lambda i: (i, 0)),
            pl.BlockSpec((_RB, NC), lambda i: (i, 0)),
        ],
        out_specs=[
            pl.BlockSpec((_RB, DD), lambda i: (i, 0)),
            pl.BlockSpec((_RB, 1), lambda i: (i, 0)),
        ],
        out_shape=[
            jax.ShapeDtypeStruct((NN, DD), jnp.float32),
            jax.ShapeDtypeStruct((NN, 1), jnp.float32),
        ],
    )(h, dgt)


def _tc_mid_body(sp_ref, hp_ref, dinv_ref, b_ref, w_ref, out_ref):
    s = sp_ref[0] + sp_ref[1] + hp_ref[...]
    z = jnp.maximum(dinv_ref[...] * s + b_ref[...], 0.0)
    h = jnp.dot(z, w_ref[...], preferred_element_type=jnp.float32)
    out_ref[...] = h * dinv_ref[...]


def _tc_mid(sp, hp, dinv, b, W2):
    return pl.pallas_call(
        _tc_mid_body,
        grid=(_GRID,),
        in_specs=[
            pl.BlockSpec((NC, _RB, DD), lambda i: (0, i, 0)),
            pl.BlockSpec((_RB, DD), lambda i: (i, 0)),
            pl.BlockSpec((_RB, 1), lambda i: (i, 0)),
            pl.BlockSpec((1, DD), lambda i: (0, 0)),
            pl.BlockSpec((DD, DD), lambda i: (0, 0)),
        ],
        out_specs=pl.BlockSpec((_RB, DD), lambda i: (i, 0)),
        out_shape=jax.ShapeDtypeStruct((NN, DD), jnp.float32),
    )(sp, hp, dinv, b, W2)


def _tc_last_body(sp_ref, hp_ref, dinv_ref, b_ref, out_ref):
    s = sp_ref[0] + sp_ref[1] + hp_ref[...]
    out_ref[...] = dinv_ref[...] * s + b_ref[...]


def _tc_last(sp, hp, dinv, b):
    return pl.pallas_call(
        _tc_last_body,
        grid=(_GRID,),
        in_specs=[
            pl.BlockSpec((NC, _RB, DD), lambda i: (0, i, 0)),
            pl.BlockSpec((_RB, DD), lambda i: (i, 0)),
            pl.BlockSpec((_RB, 1), lambda i: (i, 0)),
            pl.BlockSpec((1, DD), lambda i: (0, 0)),
        ],
        out_specs=pl.BlockSpec((_RB, DD), lambda i: (i, 0)),
        out_shape=jax.ShapeDtypeStruct((NN, DD), jnp.float32),
    )(sp, hp, dinv, b)


def kernel(x, edge_index, edge_attr, W1, b1, W2, b2):
    row = edge_index[0]
    col = edge_index[1]
    col3 = col.reshape(NW, NCHUNK, CH)
    ew3 = edge_attr.reshape(NW, NCHUNK, CH)
    degp = _sc_deg(col3, ew3)
    h1 = _tc_mm(x, W1)
    dgt = jnp.transpose(degp[:, :NN])
    hp1, dinv = _tc_scale(h1, dgt)
    s1 = _sc_msg(hp1, row, col, edge_attr)
    hp2 = _tc_mid(s1[:, :NN], hp1, dinv, b1.reshape(1, DD), W2)
    s2 = _sc_msg(hp2, row, col, edge_attr)
    out = _tc_last(s2[:, :NN], hp2, dinv, b2.reshape(1, DD))
    return out


# speedup vs baseline: 29.5558x; 1.0469x over previous
import dataclasses
import functools

import jax
import jax.numpy as jnp
from jax import lax
from jax.experimental import pallas as pl
from jax.experimental.pallas import tpu as pltpu
from jax.experimental.pallas import tpu_sc as plsc

NN = 10000
EE = 320000
DD = 128
NC = 2
NS = 16
NW = NC * NS
EPW = EE // NW
CH = 80
NCHUNK = EPW // CH
NPAD = 10240
RPT = NPAD // NS
NPADM = 10112
RPTM = NPADM // NS

_vmesh = plsc.VectorSubcoreMesh(core_axis_name="c", subcore_axis_name="s")

_sc_params = pltpu.CompilerParams()
if "needs_layout_passes" in pltpu.CompilerParams.__dataclass_fields__:
    _sc_params = dataclasses.replace(_sc_params, needs_layout_passes=False)


def _splat16(v):
    return jnp.zeros((16,), jnp.int32) + v


def _vcopy_idx(src1d, base, dst1d, n):
    q = 0
    while q + 16 <= n:
        dst1d[pl.ds(q, 16)] = src1d[pl.ds(base + q, 16)]
        q += 16
    if q < n:
        dst1d[pl.ds(n - 16, 16)] = src1d[pl.ds(base + n - 16, 16)]


def _sc_deg_body(col_hbm, ew_hbm, out_hbm, colall, ewall, zv, degsh, sem):
    cid = lax.axis_index("c")
    sid = lax.axis_index("s")
    wid = sid * NC + cid

    @pl.loop(0, RPT, step=16)
    def _zero(i):
        zv[pl.ds(i, 16)] = jnp.zeros((16,), jnp.float32)

    pltpu.sync_copy(zv, degsh.at[pl.ds(sid * RPT, RPT)])
    pltpu.sync_copy(col_hbm.at[wid], colall)
    pltpu.sync_copy(ew_hbm.at[wid], ewall)
    plsc.subcore_barrier()

    @pl.loop(0, NCHUNK // 5)
    def _chunk(g):
        descs = []
        for u in range(5):
            c = g * 5 + u
            descs.append(pltpu.async_copy(
                ewall.at[c], degsh.at[colall.at[c]], sem, add=True))
        for d in descs:
            d.wait()

    plsc.subcore_barrier()
    pltpu.sync_copy(degsh.at[pl.ds(sid * RPT, RPT)],
                    out_hbm.at[cid, pl.ds(sid * RPT, RPT)])


def _sc_deg(col3, ew3):
    k = pl.kernel(
        _sc_deg_body,
        out_type=jax.ShapeDtypeStruct((NC, NPAD), jnp.float32),
        mesh=_vmesh,
        scratch_types=[
            pltpu.VMEM((NCHUNK, CH), jnp.int32),
            pltpu.VMEM((NCHUNK, CH), jnp.float32),
            pltpu.VMEM((RPT,), jnp.float32),
            pltpu.VMEM_SHARED((NPAD,), jnp.float32),
            pltpu.SemaphoreType.DMA,
        ],
        compiler_params=_sc_params,
    )
    return k(col3, ew3)


CHM = 80
NCHM = EPW // CHM


def _sc_msg_body(hp_hbm, row_hbm, col_hbm, ew_hbm, out_hbm,
                 rowvs, colvs, ewvs, scolvs, bufs, ssh, gsems, ssems, isems):
    cid = lax.axis_index("c")
    sid = lax.axis_index("s")
    wid = sid * NC + cid
    base = wid * EPW

    for b in range(4):
        @pl.loop(0, CHM)
        def _zero(r, _b=b):
            for p in range(DD // 16):
                bufs[_b][r, pl.ds(p * 16, 16)] = jnp.zeros((16,), jnp.float32)

        for q in range(CHM // 16):
            scolvs[b][pl.ds(q * 16, 16)] = jnp.zeros((16,), jnp.int32)

    for z in range(RPTM // CHM):
        pltpu.sync_copy(bufs[0], ssh.at[pl.ds(sid * RPTM + z * CHM, CHM)])
    rem = RPTM % CHM
    if rem:
        pltpu.sync_copy(
            bufs[0].at[pl.ds(0, rem)],
            ssh.at[pl.ds(sid * RPTM + (RPTM // CHM) * CHM, rem)])
    plsc.subcore_barrier()

    def iload(c, k):
        sl = pl.ds(base + c * CHM, CHM)
        pltpu.async_copy(row_hbm.at[sl], rowvs[k], isems[k])
        pltpu.async_copy(col_hbm.at[sl], colvs[k], isems[k])
        pltpu.async_copy(ew_hbm.at[sl], ewvs[k], isems[k])

    def iwait(c, k):
        sl = pl.ds(base + c * CHM, CHM)
        pltpu.make_async_copy(row_hbm.at[sl], rowvs[k], isems[k]).wait()
        pltpu.make_async_copy(col_hbm.at[sl], colvs[k], isems[k]).wait()
        pltpu.make_async_copy(ew_hbm.at[sl], ewvs[k], isems[k]).wait()

    def gather(k):
        pltpu.async_copy(hp_hbm.at[rowvs[k]], bufs[k], gsems[k])

    def gwait(k):
        pltpu.make_async_copy(hp_hbm.at[rowvs[k]], bufs[k], gsems[k]).wait()

    def swait(k):
        pltpu.make_async_copy(bufs[k], ssh.at[scolvs[k]], ssems[k]).wait()

    def scale(buf, ewv):
        @pl.loop(0, CHM, step=16)
        def _s(j):
            wv = ewv[pl.ds(j, 16)]
            for u in range(16):
                w = jnp.take(wv, jnp.full((16,), u, jnp.int32))
                for p in range(DD // 16):
                    sl = pl.ds(p * 16, 16)
                    buf[j + u, sl] = buf[j + u, sl] * w

    def quarter(c, k, do_gather, do_iload):
        k2 = (k + 2) % 4
        gwait(k)
        scale(bufs[k], ewvs[k])
        swait(k2)
        if do_gather:
            iwait(c + 2, k2)
            gather(k2)
        _vcopy_idx(colvs[k], 0, scolvs[k], CHM)
        pltpu.async_copy(bufs[k], ssh.at[scolvs[k]], ssems[k], add=True)
        if do_iload:
            iload(c + 4, k)

    for k in range(4):
        iload(k, k)
    iwait(0, 0)
    gather(0)
    iwait(1, 1)
    gather(1)
    pltpu.async_copy(bufs[2], ssh.at[scolvs[2]], ssems[2], add=True)
    pltpu.async_copy(bufs[3], ssh.at[scolvs[3]], ssems[3], add=True)

    @pl.loop(0, NCHM - 5, step=4)
    def _chunk(c0):
        for q in range(4):
            quarter(c0 + q, q, True, True)

    for c in range(NCHM - 5, NCHM):
        quarter(c, c % 4, c + 2 < NCHM, c + 4 < NCHM)

    swait((NCHM - 2) % 4)
    swait((NCHM - 1) % 4)

    plsc.subcore_barrier()
    for z in range(RPTM // 128):
        r0 = sid * RPTM + z * 128
        pltpu.sync_copy(ssh.at[pl.ds(r0, 128)], out_hbm.at[cid, pl.ds(r0, 128)])
    remr = RPTM % 128
    if remr:
        r0 = sid * RPTM + (RPTM // 128) * 128
        pltpu.sync_copy(ssh.at[pl.ds(r0, remr)],
                        out_hbm.at[cid, pl.ds(r0, remr)])


def _sc_msg(hp, row, col, ew):
    def body(hp_hbm, row_hbm, col_hbm, ew_hbm, out_hbm,
             rv0, rv1, rv2, rv3, cv0, cv1, cv2, cv3, ev0, ev1, ev2, ev3,
             sv0, sv1, sv2, sv3, b0, b1, b2, b3, ssh,
             gs0, gs1, gs2, gs3, ss0, ss1, ss2, ss3, is0, is1, is2, is3):
        _sc_msg_body(hp_hbm, row_hbm, col_hbm, ew_hbm, out_hbm,
                     [rv0, rv1, rv2, rv3], [cv0, cv1, cv2, cv3],
                     [ev0, ev1, ev2, ev3], [sv0, sv1, sv2, sv3],
                     [b0, b1, b2, b3], ssh,
                     [gs0, gs1, gs2, gs3], [ss0, ss1, ss2, ss3],
                     [is0, is1, is2, is3])

    k = pl.kernel(
        body,
        out_type=jax.ShapeDtypeStruct((NC, NPADM, DD), jnp.float32),
        mesh=_vmesh,
        scratch_types=(
            [pltpu.VMEM((CHM,), jnp.int32) for _ in range(4)]
            + [pltpu.VMEM((CHM,), jnp.int32) for _ in range(4)]
            + [pltpu.VMEM((CHM,), jnp.float32) for _ in range(4)]
            + [pltpu.VMEM((CHM,), jnp.int32) for _ in range(4)]
            + [pltpu.VMEM((CHM, DD), jnp.float32) for _ in range(4)]
            + [pltpu.VMEM_SHARED((NPADM, DD), jnp.float32)]
            + [pltpu.SemaphoreType.DMA for _ in range(12)]
        ),
        compiler_params=_sc_params,
    )
    return k(hp, row, col, ew)


_RB = 1000
_GRID = NN // _RB


def _tc_mm_body(x_ref, w_ref, h_ref):
    h_ref[...] = jnp.dot(x_ref[...], w_ref[...],
                         preferred_element_type=jnp.float32)


def _tc_mm(x, W):
    return pl.pallas_call(
        _tc_mm_body,
        grid=(_GRID,),
        in_specs=[
            pl.BlockSpec((_RB, DD), lambda i: (i, 0)),
            pl.BlockSpec((DD, DD), lambda i: (0, 0)),
        ],
        out_specs=pl.BlockSpec((_RB, DD), lambda i: (i, 0)),
        out_shape=jax.ShapeDtypeStruct((NN, DD), jnp.float32),
    )(x, W)


def _tc_scale_body(h_ref, dgt_ref, hp_ref, dinv_ref):
    deg = 1.0 + dgt_ref[:, 0] + dgt_ref[:, 1]
    dinv = lax.rsqrt(deg)[:, None]
    dinv_ref[...] = dinv
    hp_ref[...] = h_ref[...] * dinv


def _tc_scale(h, dgt):
    return pl.pallas_call(
        _tc_scale_body,
        grid=(_GRID,),
        in_specs=[
            pl.BlockSpec((_RB, DD), lambda i: (i, 0)),
            pl.BlockSpec((_RB, NC), lambda i: (i, 0)),
        ],
        out_specs=[
            pl.BlockSpec((_RB, DD), lambda i: (i, 0)),
            pl.BlockSpec((_RB, 1), lambda i: (i, 0)),
        ],
        out_shape=[
            jax.ShapeDtypeStruct((NN, DD), jnp.float32),
            jax.ShapeDtypeStruct((NN, 1), jnp.float32),
        ],
    )(h, dgt)


def _tc_mid_body(sp_ref, hp_ref, dinv_ref, b_ref, w_ref, out_ref):
    s = sp_ref[0] + sp_ref[1] + hp_ref[...]
    z = jnp.maximum(dinv_ref[...] * s + b_ref[...], 0.0)
    h = jnp.dot(z, w_ref[...], preferred_element_type=jnp.float32)
    out_ref[...] = h * dinv_ref[...]


def _tc_mid(sp, hp, dinv, b, W2):
    return pl.pallas_call(
        _tc_mid_body,
        grid=(_GRID,),
        in_specs=[
            pl.BlockSpec((NC, _RB, DD), lambda i: (0, i, 0)),
            pl.BlockSpec((_RB, DD), lambda i: (i, 0)),
            pl.BlockSpec((_RB, 1), lambda i: (i, 0)),
            pl.BlockSpec((1, DD), lambda i: (0, 0)),
            pl.BlockSpec((DD, DD), lambda i: (0, 0)),
        ],
        out_specs=pl.BlockSpec((_RB, DD), lambda i: (i, 0)),
        out_shape=jax.ShapeDtypeStruct((NN, DD), jnp.float32),
    )(sp, hp, dinv, b, W2)


def _tc_last_body(sp_ref, hp_ref, dinv_ref, b_ref, out_ref):
    s = sp_ref[0] + sp_ref[1] + hp_ref[...]
    out_ref[...] = dinv_ref[...] * s + b_ref[...]


def _tc_last(sp, hp, dinv, b):
    return pl.pallas_call(
        _tc_last_body,
        grid=(_GRID,),
        in_specs=[
            pl.BlockSpec((NC, _RB, DD), lambda i: (0, i, 0)),
            pl.BlockSpec((_RB, DD), lambda i: (i, 0)),
            pl.BlockSpec((_RB, 1), lambda i: (i, 0)),
            pl.BlockSpec((1, DD), lambda i: (0, 0)),
        ],
        out_specs=pl.BlockSpec((_RB, DD), lambda i: (i, 0)),
        out_shape=jax.ShapeDtypeStruct((NN, DD), jnp.float32),
    )(sp, hp, dinv, b)


def kernel(x, edge_index, edge_attr, W1, b1, W2, b2):
    row = edge_index[0]
    col = edge_index[1]
    col3 = col.reshape(NW, NCHUNK, CH)
    ew3 = edge_attr.reshape(NW, NCHUNK, CH)
    degp = _sc_deg(col3, ew3)
    h1 = _tc_mm(x, W1)
    dgt = jnp.transpose(degp[:, :NN])
    hp1, dinv = _tc_scale(h1, dgt)
    s1 = _sc_msg(hp1, row, col, edge_attr)
    hp2 = _tc_mid(s1[:, :NN], hp1, dinv, b1.reshape(1, DD), W2)
    s2 = _sc_msg(hp2, row, col, edge_attr)
    out = _tc_last(s2[:, :NN], hp2, dinv, b2.reshape(1, DD))
    return out


# speedup vs baseline: 30.8744x; 1.0446x over previous
import dataclasses
import functools

import jax
import jax.numpy as jnp
from jax import lax
from jax.experimental import pallas as pl
from jax.experimental.pallas import tpu as pltpu
from jax.experimental.pallas import tpu_sc as plsc

NN = 10000
EE = 320000
DD = 128
NC = 2
NS = 16
NW = NC * NS
EPW = EE // NW
CH = 80
NCHUNK = EPW // CH
NPAD = 10240
RPT = NPAD // NS
NPADM = 10112
RPTM = NPADM // NS

_vmesh = plsc.VectorSubcoreMesh(core_axis_name="c", subcore_axis_name="s")

_sc_params = pltpu.CompilerParams()
if "needs_layout_passes" in pltpu.CompilerParams.__dataclass_fields__:
    _sc_params = dataclasses.replace(_sc_params, needs_layout_passes=False)


def _splat16(v):
    return jnp.zeros((16,), jnp.int32) + v


def _vcopy_idx(src1d, base, dst1d, n):
    q = 0
    while q + 16 <= n:
        dst1d[pl.ds(q, 16)] = src1d[pl.ds(base + q, 16)]
        q += 16
    if q < n:
        dst1d[pl.ds(n - 16, 16)] = src1d[pl.ds(base + n - 16, 16)]


def _sc_deg_body(col_hbm, ew_hbm, out_hbm, colall, ewall, zv, degsh, sem):
    cid = lax.axis_index("c")
    sid = lax.axis_index("s")
    wid = sid * NC + cid

    @pl.loop(0, RPT, step=16)
    def _zero(i):
        zv[pl.ds(i, 16)] = jnp.zeros((16,), jnp.float32)

    pltpu.sync_copy(zv, degsh.at[pl.ds(sid * RPT, RPT)])
    pltpu.sync_copy(col_hbm.at[wid], colall)
    pltpu.sync_copy(ew_hbm.at[wid], ewall)
    plsc.subcore_barrier()

    @pl.loop(0, NCHUNK // 5)
    def _chunk(g):
        descs = []
        for u in range(5):
            c = g * 5 + u
            descs.append(pltpu.async_copy(
                ewall.at[c], degsh.at[colall.at[c]], sem, add=True))
        for d in descs:
            d.wait()

    plsc.subcore_barrier()
    pltpu.sync_copy(degsh.at[pl.ds(sid * RPT, RPT)],
                    out_hbm.at[cid, pl.ds(sid * RPT, RPT)])


def _sc_deg(col3, ew3):
    k = pl.kernel(
        _sc_deg_body,
        out_type=jax.ShapeDtypeStruct((NC, NPAD), jnp.float32),
        mesh=_vmesh,
        scratch_types=[
            pltpu.VMEM((NCHUNK, CH), jnp.int32),
            pltpu.VMEM((NCHUNK, CH), jnp.float32),
            pltpu.VMEM((RPT,), jnp.float32),
            pltpu.VMEM_SHARED((NPAD,), jnp.float32),
            pltpu.SemaphoreType.DMA,
        ],
        compiler_params=_sc_params,
    )
    return k(col3, ew3)


CHM = 80
NCHM = EPW // CHM


def _sc_msg_body(hp_hbm, row_hbm, col_hbm, ew_hbm, out_hbm,
                 rowvs, colvs, ewvs, scolvs, bufs, ssh, gsems, ssems, isems):
    cid = lax.axis_index("c")
    sid = lax.axis_index("s")
    wid = sid * NC + cid
    base = wid * EPW

    for b in range(4):
        @pl.loop(0, CHM)
        def _zero(r, _b=b):
            for p in range(DD // 16):
                bufs[_b][r, pl.ds(p * 16, 16)] = jnp.zeros((16,), jnp.float32)

        for q in range(CHM // 16):
            scolvs[b][pl.ds(q * 16, 16)] = jnp.zeros((16,), jnp.int32)

    for z in range(RPTM // CHM):
        pltpu.sync_copy(bufs[0], ssh.at[pl.ds(sid * RPTM + z * CHM, CHM)])
    rem = RPTM % CHM
    if rem:
        pltpu.sync_copy(
            bufs[0].at[pl.ds(0, rem)],
            ssh.at[pl.ds(sid * RPTM + (RPTM // CHM) * CHM, rem)])
    plsc.subcore_barrier()

    def iload(c, k):
        sl = pl.ds(base + c * CHM, CHM)
        pltpu.async_copy(row_hbm.at[sl], rowvs[k], isems[k])
        pltpu.async_copy(col_hbm.at[sl], colvs[k], isems[k])
        pltpu.async_copy(ew_hbm.at[sl], ewvs[k], isems[k])

    def iwait(c, k):
        sl = pl.ds(base + c * CHM, CHM)
        pltpu.make_async_copy(row_hbm.at[sl], rowvs[k], isems[k]).wait()
        pltpu.make_async_copy(col_hbm.at[sl], colvs[k], isems[k]).wait()
        pltpu.make_async_copy(ew_hbm.at[sl], ewvs[k], isems[k]).wait()

    def gather(k):
        pltpu.async_copy(hp_hbm.at[rowvs[k]], bufs[k], gsems[k])

    def gwait(k):
        pltpu.make_async_copy(hp_hbm.at[rowvs[k]], bufs[k], gsems[k]).wait()

    def swait(k):
        pltpu.make_async_copy(bufs[k], ssh.at[scolvs[k]], ssems[k]).wait()

    def scale(buf, ewv):
        @pl.loop(0, CHM, step=16)
        def _s(j):
            wv = ewv[pl.ds(j, 16)]
            for u in range(16):
                w = jnp.take(wv, jnp.full((16,), u, jnp.int32))
                for p in range(DD // 16):
                    sl = pl.ds(p * 16, 16)
                    buf[j + u, sl] = buf[j + u, sl] * w

    def quarter(c, k, do_gather, do_iload):
        k2 = (k + 2) % 4
        gwait(k)
        scale(bufs[k], ewvs[k])
        swait(k2)
        if do_gather:
            iwait(c + 2, k2)
            gather(k2)
        _vcopy_idx(colvs[k], 0, scolvs[k], CHM)
        pltpu.async_copy(bufs[k], ssh.at[scolvs[k]], ssems[k], add=True)
        if do_iload:
            iload(c + 4, k)

    for k in range(4):
        iload(k, k)
    iwait(0, 0)
    gather(0)
    iwait(1, 1)
    gather(1)
    pltpu.async_copy(bufs[2], ssh.at[scolvs[2]], ssems[2], add=True)
    pltpu.async_copy(bufs[3], ssh.at[scolvs[3]], ssems[3], add=True)

    @pl.loop(0, NCHM - 5, step=4)
    def _chunk(c0):
        for q in range(4):
            quarter(c0 + q, q, True, True)

    for c in range(NCHM - 5, NCHM):
        quarter(c, c % 4, c + 2 < NCHM, c + 4 < NCHM)

    swait((NCHM - 2) % 4)
    swait((NCHM - 1) % 4)

    plsc.subcore_barrier()
    for z in range(RPTM // 128):
        r0 = sid * RPTM + z * 128
        pltpu.sync_copy(ssh.at[pl.ds(r0, 128)], out_hbm.at[cid, pl.ds(r0, 128)])
    remr = RPTM % 128
    if remr:
        r0 = sid * RPTM + (RPTM // 128) * 128
        pltpu.sync_copy(ssh.at[pl.ds(r0, remr)],
                        out_hbm.at[cid, pl.ds(r0, remr)])


def _sc_msg(hp, row, col, ew):
    def body(hp_hbm, row_hbm, col_hbm, ew_hbm, out_hbm,
             rv0, rv1, rv2, rv3, cv0, cv1, cv2, cv3, ev0, ev1, ev2, ev3,
             sv0, sv1, sv2, sv3, b0, b1, b2, b3, ssh,
             gs0, gs1, gs2, gs3, ss0, ss1, ss2, ss3, is0, is1, is2, is3):
        _sc_msg_body(hp_hbm, row_hbm, col_hbm, ew_hbm, out_hbm,
                     [rv0, rv1, rv2, rv3], [cv0, cv1, cv2, cv3],
                     [ev0, ev1, ev2, ev3], [sv0, sv1, sv2, sv3],
                     [b0, b1, b2, b3], ssh,
                     [gs0, gs1, gs2, gs3], [ss0, ss1, ss2, ss3],
                     [is0, is1, is2, is3])

    k = pl.kernel(
        body,
        out_type=jax.ShapeDtypeStruct((NC, NPADM, DD), jnp.float32),
        mesh=_vmesh,
        scratch_types=(
            [pltpu.VMEM((CHM,), jnp.int32) for _ in range(4)]
            + [pltpu.VMEM((CHM,), jnp.int32) for _ in range(4)]
            + [pltpu.VMEM((CHM,), jnp.float32) for _ in range(4)]
            + [pltpu.VMEM((CHM,), jnp.int32) for _ in range(4)]
            + [pltpu.VMEM((CHM, DD), jnp.float32) for _ in range(4)]
            + [pltpu.VMEM_SHARED((NPADM, DD), jnp.float32)]
            + [pltpu.SemaphoreType.DMA for _ in range(12)]
        ),
        compiler_params=_sc_params,
    )
    return k(hp, row, col, ew)


_RB = 1000
_GRID = NN // _RB


def _tc_first_body(x_ref, w_ref, dgt_ref, hp_ref, dinv_ref):
    deg = 1.0 + dgt_ref[:, 0] + dgt_ref[:, 1]
    dinv = lax.rsqrt(deg)[:, None]
    dinv_ref[...] = dinv
    h = jnp.dot(x_ref[...], w_ref[...], preferred_element_type=jnp.float32)
    hp_ref[...] = h * dinv


def _tc_first(x, W, dgt):
    return pl.pallas_call(
        _tc_first_body,
        grid=(_GRID,),
        in_specs=[
            pl.BlockSpec((_RB, DD), lambda i: (i, 0)),
            pl.BlockSpec((DD, DD), lambda i: (0, 0)),
            pl.BlockSpec((_RB, NC), lambda i: (i, 0)),
        ],
        out_specs=[
            pl.BlockSpec((_RB, DD), lambda i: (i, 0)),
            pl.BlockSpec((_RB, 1), lambda i: (i, 0)),
        ],
        out_shape=[
            jax.ShapeDtypeStruct((NN, DD), jnp.float32),
            jax.ShapeDtypeStruct((NN, 1), jnp.float32),
        ],
    )(x, W, dgt)


def _tc_mid_body(sp_ref, hp_ref, dinv_ref, b_ref, w_ref, out_ref):
    s = sp_ref[0] + sp_ref[1] + hp_ref[...]
    z = jnp.maximum(dinv_ref[...] * s + b_ref[...], 0.0)
    h = jnp.dot(z, w_ref[...], preferred_element_type=jnp.float32)
    out_ref[...] = h * dinv_ref[...]


def _tc_mid(sp, hp, dinv, b, W2):
    return pl.pallas_call(
        _tc_mid_body,
        grid=(_GRID,),
        in_specs=[
            pl.BlockSpec((NC, _RB, DD), lambda i: (0, i, 0)),
            pl.BlockSpec((_RB, DD), lambda i: (i, 0)),
            pl.BlockSpec((_RB, 1), lambda i: (i, 0)),
            pl.BlockSpec((1, DD), lambda i: (0, 0)),
            pl.BlockSpec((DD, DD), lambda i: (0, 0)),
        ],
        out_specs=pl.BlockSpec((_RB, DD), lambda i: (i, 0)),
        out_shape=jax.ShapeDtypeStruct((NN, DD), jnp.float32),
    )(sp, hp, dinv, b, W2)


def _tc_last_body(sp_ref, hp_ref, dinv_ref, b_ref, out_ref):
    s = sp_ref[0] + sp_ref[1] + hp_ref[...]
    out_ref[...] = dinv_ref[...] * s + b_ref[...]


def _tc_last(sp, hp, dinv, b):
    return pl.pallas_call(
        _tc_last_body,
        grid=(_GRID,),
        in_specs=[
            pl.BlockSpec((NC, _RB, DD), lambda i: (0, i, 0)),
            pl.BlockSpec((_RB, DD), lambda i: (i, 0)),
            pl.BlockSpec((_RB, 1), lambda i: (i, 0)),
            pl.BlockSpec((1, DD), lambda i: (0, 0)),
        ],
        out_specs=pl.BlockSpec((_RB, DD), lambda i: (i, 0)),
        out_shape=jax.ShapeDtypeStruct((NN, DD), jnp.float32),
    )(sp, hp, dinv, b)


def kernel(x, edge_index, edge_attr, W1, b1, W2, b2):
    row = edge_index[0]
    col = edge_index[1]
    col3 = col.reshape(NW, NCHUNK, CH)
    ew3 = edge_attr.reshape(NW, NCHUNK, CH)
    degp = _sc_deg(col3, ew3)
    dgt = jnp.transpose(degp[:, :NN])
    hp1, dinv = _tc_first(x, W1, dgt)
    s1 = _sc_msg(hp1, row, col, edge_attr)
    hp2 = _tc_mid(s1, hp1, dinv, b1.reshape(1, DD), W2)
    s2 = _sc_msg(hp2, row, col, edge_attr)
    out = _tc_last(s2, hp2, dinv, b2.reshape(1, DD))
    return out
